# baseline (speedup 1.0000x reference)
import jax
import jax.numpy as jnp
from jax import lax
from jax.experimental import pallas as pl
from jax.experimental.pallas import tpu as pltpu
from jax.experimental.pallas import tpu_sc as plsc

NC, NS = 2, 16
NW = NC * NS
B = 4096 * 200
D_I32 = 32
B_PER_W = B // NW
CHUNK = 1024
NCHUNK = B_PER_W // CHUNK


def _gather_body(idx_hbm, tab_hbm, out_hbm, idx_v, rows_v, sem):
    wid = lax.axis_index("s") * NC + lax.axis_index("c")
    base = wid * B_PER_W

    def step(i, carry):
        off = base + i * CHUNK
        pltpu.sync_copy(idx_hbm.at[pl.ds(off, CHUNK)], idx_v)
        pltpu.async_copy(tab_hbm.at[idx_v], rows_v, sem).wait()
        pltpu.sync_copy(rows_v, out_hbm.at[pl.ds(off, CHUNK)])
        return carry

    lax.fori_loop(0, NCHUNK, step, 0)


def kernel(input_ids, weight):
    idx = input_ids.reshape(-1).astype(jnp.int32)
    tab = lax.bitcast_convert_type(
        weight.reshape(weight.shape[0], D_I32, 2), jnp.int32)

    run = pl.kernel(
        _gather_body,
        out_type=jax.ShapeDtypeStruct((B, D_I32), jnp.int32),
        mesh=plsc.VectorSubcoreMesh(
            core_axis_name="c", subcore_axis_name="s",
            num_cores=NC, num_subcores=NS),
        scratch_types=[
            pltpu.VMEM((CHUNK,), jnp.int32),
            pltpu.VMEM((CHUNK, D_I32), jnp.int32),
            pltpu.SemaphoreType.DMA,
        ],
        compiler_params=pltpu.CompilerParams(use_tc_tiling_on_sc=False),
    )
    out = run(idx, tab)
    hidden = lax.bitcast_convert_type(out, jnp.bfloat16)
    return hidden.reshape(input_ids.shape[0], input_ids.shape[1], 64)


# speedup vs baseline: 1.0068x; 1.0068x over previous
import jax
import jax.numpy as jnp
from jax import lax
from jax.experimental import pallas as pl
from jax.experimental.pallas import tpu as pltpu
from jax.experimental.pallas import tpu_sc as plsc

NC, NS = 2, 16
NW = NC * NS
B = 4096 * 200
D_I32 = 32
B_PER_W = B // NW
CHUNK = 1600
NCHUNK = B_PER_W // CHUNK
NBUF = 2
NGROUP = NCHUNK // NBUF


def _gather_body(idx_hbm, tab_hbm, out_hbm,
                 idx0, idx1, rows0, rows1,
                 is0, is1, gs0, gs1, os0, os1):
    wid = lax.axis_index("s") * NC + lax.axis_index("c")
    base = wid * B_PER_W
    idx_v = (idx0, idx1)
    rows_v = (rows0, rows1)
    isem = (is0, is1)
    gsem = (gs0, gs1)
    osem = (os0, os1)

    def fire_idx(off, b):
        pltpu.async_copy(idx_hbm.at[pl.ds(off, CHUNK)], idx_v[b], isem[b])

    def wait_idx(off, b):
        pltpu.make_async_copy(idx_hbm.at[pl.ds(off, CHUNK)], idx_v[b],
                              isem[b]).wait()

    def wait_out(off, b):
        pltpu.make_async_copy(rows_v[b], out_hbm.at[pl.ds(off, CHUNK)],
                              osem[b]).wait()

    def gather(b):
        pltpu.async_copy(tab_hbm.at[idx_v[b]], rows_v[b], gsem[b]).wait()

    def fire_out(off, b):
        pltpu.async_copy(rows_v[b], out_hbm.at[pl.ds(off, CHUNK)], osem[b])

    for b in range(NBUF):
        fire_idx(base + b * CHUNK, b)
    for b in range(NBUF):
        off = base + b * CHUNK
        wait_idx(off, b)
        gather(b)
        fire_idx(off + NBUF * CHUNK, b)
        fire_out(off, b)

    def group(g, carry):
        i0 = g * NBUF
        for b in range(NBUF):
            off = base + (i0 + b) * CHUNK
            wait_out(off, b)
            wait_idx(off, b)
            gather(b)
            fire_idx(off + NBUF * CHUNK, b)
            fire_out(off, b)
        return carry

    lax.fori_loop(1, NGROUP - 1, group, 0)

    for b in range(NBUF):
        off = base + ((NGROUP - 1) * NBUF + b) * CHUNK
        wait_out(off, b)
        wait_idx(off, b)
        gather(b)
        fire_out(off, b)

    for b in range(NBUF):
        wait_out(base + b * CHUNK, b)


def kernel(input_ids, weight):
    idx = input_ids.reshape(-1).astype(jnp.int32)
    tab = lax.bitcast_convert_type(
        weight.reshape(weight.shape[0], D_I32, 2), jnp.int32)

    run = pl.kernel(
        _gather_body,
        out_type=jax.ShapeDtypeStruct((B, D_I32), jnp.int32),
        mesh=plsc.VectorSubcoreMesh(
            core_axis_name="c", subcore_axis_name="s",
            num_cores=NC, num_subcores=NS),
        scratch_types=[
            pltpu.VMEM((CHUNK,), jnp.int32),
            pltpu.VMEM((CHUNK,), jnp.int32),
            pltpu.VMEM((CHUNK, D_I32), jnp.int32),
            pltpu.VMEM((CHUNK, D_I32), jnp.int32),
            pltpu.SemaphoreType.DMA,
            pltpu.SemaphoreType.DMA,
            pltpu.SemaphoreType.DMA,
            pltpu.SemaphoreType.DMA,
            pltpu.SemaphoreType.DMA,
            pltpu.SemaphoreType.DMA,
        ],
        compiler_params=pltpu.CompilerParams(use_tc_tiling_on_sc=False),
    )
    out = run(idx, tab)
    hidden = lax.bitcast_convert_type(out, jnp.bfloat16)
    return hidden.reshape(input_ids.shape[0], input_ids.shape[1], 64)
